# Initial kernel scaffold; baseline (speedup 1.0000x reference)
import jax
import jax.numpy as jnp
from jax.experimental import pallas as pl


def kernel(seq, emb_table):
    raise NotImplementedError("write your pallas kernel here")


# speedup vs baseline: 4.2547x; 4.2547x over previous
import functools

import jax
import jax.numpy as jnp
from jax.experimental import pallas as pl

_BLOCK_S = 256


def _gelu_tile_kernel(table_ref, out_ref, *, batch: int):
    x = table_ref[...]
    y = 0.5 * x * (1.0 + jax.lax.erf(x * (2.0 ** -0.5)))
    out_ref[...] = jnp.concatenate([y] * batch, axis=1)


def kernel(seq, emb_table):
    seq_len, batch = seq.shape
    hidden = emb_table.shape[1]
    grid = seq_len // _BLOCK_S

    out2d = pl.pallas_call(
        functools.partial(_gelu_tile_kernel, batch=batch),
        grid=(grid,),
        in_specs=[pl.BlockSpec((_BLOCK_S, hidden), lambda i: (i, 0))],
        out_specs=pl.BlockSpec((_BLOCK_S, batch * hidden), lambda i: (i, 0)),
        out_shape=jax.ShapeDtypeStruct((seq_len, batch * hidden), emb_table.dtype),
    )(emb_table)
    return out2d.reshape(seq_len, batch, hidden)


# speedup vs baseline: 4.3408x; 1.0202x over previous
import functools

import jax
import jax.numpy as jnp
from jax.experimental import pallas as pl

_BLOCK_S = 512


def _gelu_tile_kernel(table_ref, out_ref, *, batch: int):
    x = table_ref[...]
    y = 0.5 * x * (1.0 + jax.lax.erf(x * (2.0 ** -0.5)))
    out_ref[...] = jnp.concatenate([y] * batch, axis=1)


def kernel(seq, emb_table):
    seq_len, batch = seq.shape
    hidden = emb_table.shape[1]
    grid = seq_len // _BLOCK_S

    out2d = pl.pallas_call(
        functools.partial(_gelu_tile_kernel, batch=batch),
        grid=(grid,),
        in_specs=[pl.BlockSpec((_BLOCK_S, hidden), lambda i: (i, 0))],
        out_specs=pl.BlockSpec((_BLOCK_S, batch * hidden), lambda i: (i, 0)),
        out_shape=jax.ShapeDtypeStruct((seq_len, batch * hidden), emb_table.dtype),
    )(emb_table)
    return out2d.reshape(seq_len, batch, hidden)
